# Initial kernel scaffold; baseline (speedup 1.0000x reference)
import jax
import jax.numpy as jnp
from jax.experimental import pallas as pl


def kernel(x):
    raise NotImplementedError("write your pallas kernel here")


# speedup vs baseline: 1.0046x; 1.0046x over previous
import functools

import jax
import jax.numpy as jnp
from jax import lax
from jax.experimental import pallas as pl
from jax.experimental.pallas import tpu as pltpu
from jax.experimental.pallas import tpu_sc as plsc

_PATCH = 16
_KER = 22
_PAD = 6
_NP = 32
_L = _NP * _NP
_TOP = 512
_D = 3 * _KER * _KER
_MIN_OBJ = -1000000.0


def _sort_body(sums_ref, idx_ref, gidx_ref, mask_ref, coord_ref):
    b = pl.program_id(0)
    s = sums_ref[0, 0, :]
    si = s[:, None]
    sj = s[None, :]
    ii = lax.broadcasted_iota(jnp.int32, (_L, _L), 0)
    jj = lax.broadcasted_iota(jnp.int32, (_L, _L), 1)
    before = (sj > si) | ((sj == si) & (jj < ii))
    rank = jnp.sum(before.astype(jnp.float32), axis=1).astype(jnp.int32)
    onehot = rank[:, None] == jj
    sidx_full = jnp.sum(jnp.where(onehot, ii, 0), axis=0)
    ssum_full = jnp.sum(jnp.where(onehot, s[:, None], 0.0), axis=0)
    sidx = sidx_full[:_TOP]
    ssum = ssum_full[:_TOP]
    idx_ref[0, 0, :] = sidx
    gidx_ref[0, 0, :] = sidx + b * _L
    mask_ref[0, 0, :] = (ssum > _MIN_OBJ).astype(jnp.float32)
    coord_ref[0, 0, :] = sidx // _NP
    coord_ref[0, 1, :] = sidx % _NP


_sort_call = pl.pallas_call(
    _sort_body,
    grid=(16,),
    in_specs=[pl.BlockSpec((1, 1, _L), lambda b: (b, 0, 0))],
    out_specs=[
        pl.BlockSpec((1, 1, _TOP), lambda b: (b, 0, 0)),
        pl.BlockSpec((1, 1, _TOP), lambda b: (b, 0, 0)),
        pl.BlockSpec((1, 1, _TOP), lambda b: (b, 0, 0)),
        pl.BlockSpec((1, 2, _TOP), lambda b: (b, 0, 0)),
    ],
    out_shape=[
        jax.ShapeDtypeStruct((16, 1, _TOP), jnp.int32),
        jax.ShapeDtypeStruct((16, 1, _TOP), jnp.int32),
        jax.ShapeDtypeStruct((16, 1, _TOP), jnp.float32),
        jax.ShapeDtypeStruct((16, 2, _TOP), jnp.int32),
    ],
)


_NW = 32
_FPW = (_D + _NW - 1) // _NW
_B16 = 16 * _L
_OUTW = 16 * _TOP


def _sc_gather_body(pt_hbm, gidx_hbm, out_hbm, plane_v, out_v, idx_v):
    wid = lax.axis_index("s") * 2 + lax.axis_index("c")
    pltpu.sync_copy(gidx_hbm, idx_v)
    lo = wid * _FPW

    def body(k, carry):
        f = lo + k

        @pl.when(f < _D)
        def _():
            pltpu.sync_copy(pt_hbm.at[f], plane_v)
            for ch in range(_OUTW // 16):
                ip = idx_v[ch // 8, pl.ds((ch % 8) * 16, 16)]
                hi = lax.shift_right_logical(ip, 7)
                lo_ = lax.bitwise_and(ip, 127)
                vals = plsc.load_gather(plane_v, [hi, lo_])
                out_v[ch // 8, pl.ds((ch % 8) * 16, 16)] = vals
            pltpu.sync_copy(out_v, out_hbm.at[f])

        return carry

    lax.fori_loop(0, _FPW, body, 0)


_sc_gather = functools.partial(
    pl.kernel,
    mesh=plsc.VectorSubcoreMesh(core_axis_name="c", subcore_axis_name="s"),
    out_type=jax.ShapeDtypeStruct((_D, _OUTW // 128, 128), jnp.float32),
    scratch_types=[
        pltpu.VMEM((_B16 // 128, 128), jnp.float32),
        pltpu.VMEM((_OUTW // 128, 128), jnp.float32),
        pltpu.VMEM((_OUTW // 128, 128), jnp.int32),
    ],
    compiler_params=pltpu.CompilerParams(needs_layout_passes=False),
)(_sc_gather_body)


def kernel(x):
    B, C, H, W = x.shape
    xp = jnp.pad(x, ((0, 0), (0, 0), (_PAD, _PAD), (_PAD, _PAD)))
    ii = (jnp.arange(_NP) * _PATCH)[:, None] + jnp.arange(_KER)[None, :]
    jj = (jnp.arange(_NP) * _PATCH)[:, None] + jnp.arange(_KER)[None, :]
    rows = ii[:, None, :, None]
    cols = jj[None, :, None, :]
    patches = xp[:, :, rows, cols]
    patches = patches.transpose(0, 2, 3, 1, 4, 5).reshape(B, _L, _D)
    sums = patches.sum(axis=-1)

    pt = patches.transpose(2, 0, 1).reshape(_D, _B16 // 128, 128)

    sidx3, gidx3, mask3, coord3 = _sort_call(sums.reshape(B, 1, _L))
    gidx = gidx3.reshape(_OUTW // 128, 128)

    out = _sc_gather(pt, gidx)

    patches_out = out.reshape(3, _KER, _KER, B, _TOP).transpose(3, 4, 0, 1, 2)
    mask = mask3.reshape(B, _TOP)
    coord_idx = coord3.transpose(0, 2, 1)
    return patches_out, mask, coord_idx


# speedup vs baseline: 12.7628x; 12.7040x over previous
import functools

import jax
import jax.numpy as jnp
import numpy as np
from jax import lax
from jax.experimental import pallas as pl
from jax.experimental.pallas import tpu as pltpu
from jax.experimental.pallas import tpu_sc as plsc

_PATCH = 16
_KER = 22
_PAD = 6
_NP = 32
_L = _NP * _NP
_TOP = 512
_D = 3 * _KER * _KER
_MIN_OBJ = -1000000.0


_CHUNK = _NP * 484
_FT = _CHUNK // 128

def _unfold_index_tables():
    n = np.arange(_CHUNK)
    pw = n // 484
    r = n % 484
    kh = r // _KER
    j = r % _KER
    col = pw * _PATCH + j - _PAD
    rows = np.where(col < 0, 1000, kh + 2)
    cols = np.clip(col, 0, 511)
    return rows.astype(np.int32), cols.astype(np.int32)

_IDXR_NP, _IDXC_NP = _unfold_index_tables()


def _sc_unfold_body(x_hbm, idxr_hbm, idxc_hbm, out_hbm, stage_v, outb_v,
                    idxr_v, idxc_v):
    b = lax.axis_index("s")
    half = lax.axis_index("c")
    tr = b // 8
    br = b % 8
    pltpu.sync_copy(idxr_hbm, idxr_v)
    pltpu.sync_copy(idxc_hbm, idxc_v)
    zero16 = jnp.zeros((16,), jnp.float32)
    for rr in range(24, 32):
        for q in range(512 // 16):
            stage_v[rr, pl.ds(q * 16, 16)] = zero16

    def body(k, carry):
        c = k // 16
        ph = half * 16 + (k % 16)
        start = jnp.where(ph == 0, 0, ph * _PATCH - 8)
        adj = jnp.where(ph == 0, 8, 0).astype(jnp.int32)
        pltpu.sync_copy(x_hbm.at[b, c, pl.ds(start, 24)],
                        stage_v.at[pl.ds(0, 24)])
        for n in range(_CHUNK // 16):
            rv = idxr_v[pl.ds(n * 16, 16)] - adj
            rv = jnp.where((rv < 0) | (rv > 23), 24, rv)
            cv = idxc_v[pl.ds(n * 16, 16)]
            outb_v[n // 8, pl.ds((n % 8) * 16, 16)] = plsc.load_gather(
                stage_v, [rv, cv])
        pltpu.sync_copy(outb_v, out_hbm.at[c, tr, pl.ds(ph * _FT, _FT), br])
        return carry

    lax.fori_loop(0, 48, body, 0)


_sc_unfold = functools.partial(
    pl.kernel,
    mesh=plsc.VectorSubcoreMesh(core_axis_name="c", subcore_axis_name="s"),
    out_type=jax.ShapeDtypeStruct((3, 2, 3872, 8, 128), jnp.float32),
    scratch_types=[
        pltpu.VMEM((32, 512), jnp.float32),
        pltpu.VMEM((_FT, 128), jnp.float32),
        pltpu.VMEM((_CHUNK,), jnp.int32),
        pltpu.VMEM((_CHUNK,), jnp.int32),
    ],
    compiler_params=pltpu.CompilerParams(needs_layout_passes=False),
)(_sc_unfold_body)


def _sort_body(sums_ref, idx_ref, gidx_ref, mask_ref, coord_ref):
    b = pl.program_id(0)
    s = sums_ref[0, 0, :]
    si = s[:, None]
    sj = s[None, :]
    ii = lax.broadcasted_iota(jnp.int32, (_L, _L), 0)
    jj = lax.broadcasted_iota(jnp.int32, (_L, _L), 1)
    before = (sj > si) | ((sj == si) & (jj < ii))
    rank = jnp.sum(before.astype(jnp.float32), axis=1).astype(jnp.int32)
    onehot = rank[:, None] == jj
    sidx_full = jnp.sum(jnp.where(onehot, ii, 0), axis=0)
    ssum_full = jnp.sum(jnp.where(onehot, s[:, None], 0.0), axis=0)
    sidx = sidx_full[:_TOP]
    ssum = ssum_full[:_TOP]
    idx_ref[0, 0, :] = sidx
    gidx_ref[0, 0, :] = sidx + b * _L
    mask_ref[0, 0, :] = (ssum > _MIN_OBJ).astype(jnp.float32)
    coord_ref[0, 0, :] = sidx // _NP
    coord_ref[0, 1, :] = sidx % _NP


_sort_call = pl.pallas_call(
    _sort_body,
    grid=(16,),
    in_specs=[pl.BlockSpec((1, 1, _L), lambda b: (b, 0, 0))],
    out_specs=[
        pl.BlockSpec((1, 1, _TOP), lambda b: (b, 0, 0)),
        pl.BlockSpec((1, 1, _TOP), lambda b: (b, 0, 0)),
        pl.BlockSpec((1, 1, _TOP), lambda b: (b, 0, 0)),
        pl.BlockSpec((1, 2, _TOP), lambda b: (b, 0, 0)),
    ],
    out_shape=[
        jax.ShapeDtypeStruct((16, 1, _TOP), jnp.int32),
        jax.ShapeDtypeStruct((16, 1, _TOP), jnp.int32),
        jax.ShapeDtypeStruct((16, 1, _TOP), jnp.float32),
        jax.ShapeDtypeStruct((16, 2, _TOP), jnp.int32),
    ],
)


_NW = 32
_FPW = (_D + _NW - 1) // _NW
_B16 = 16 * _L
_OUTW = 16 * _TOP


def _sc_gather_body(pt_hbm, gidx_hbm, out_hbm, plane_v, out_v, idx_v):
    wid = lax.axis_index("s") * 2 + lax.axis_index("c")
    pltpu.sync_copy(gidx_hbm, idx_v)
    lo = wid * _FPW

    def body(k, carry):
        f = lo + k

        @pl.when(f < _D)
        def _():
            pltpu.sync_copy(pt_hbm.at[f], plane_v)
            for ch in range(_OUTW // 16):
                ip = idx_v[ch // 8, pl.ds((ch % 8) * 16, 16)]
                hi = lax.shift_right_logical(ip, 7)
                lo_ = lax.bitwise_and(ip, 127)
                vals = plsc.load_gather(plane_v, [hi, lo_])
                out_v[ch // 8, pl.ds((ch % 8) * 16, 16)] = vals
            pltpu.sync_copy(out_v, out_hbm.at[f])

        return carry

    lax.fori_loop(0, _FPW, body, 0)


_sc_gather = functools.partial(
    pl.kernel,
    mesh=plsc.VectorSubcoreMesh(core_axis_name="c", subcore_axis_name="s"),
    out_type=jax.ShapeDtypeStruct((_D, _OUTW // 128, 128), jnp.float32),
    scratch_types=[
        pltpu.VMEM((_B16 // 128, 128), jnp.float32),
        pltpu.VMEM((_OUTW // 128, 128), jnp.float32),
        pltpu.VMEM((_OUTW // 128, 128), jnp.int32),
    ],
    compiler_params=pltpu.CompilerParams(needs_layout_passes=False),
)(_sc_gather_body)


def kernel(x):
    B, C, H, W = x.shape
    out5 = _sc_unfold(x, jnp.asarray(_IDXR_NP), jnp.asarray(_IDXC_NP))
    u = out5.transpose(0, 1, 3, 2, 4).reshape(3, B, _L * 484)
    patches = (u.reshape(3, B, _NP, _NP, _KER, _KER)
                .transpose(1, 2, 3, 0, 4, 5).reshape(B, _L, _D))
    sums = patches.sum(axis=-1)

    pt = patches.transpose(2, 0, 1).reshape(_D, _B16 // 128, 128)

    sidx3, gidx3, mask3, coord3 = _sort_call(sums.reshape(B, 1, _L))
    gidx = gidx3.reshape(_OUTW // 128, 128)

    out = _sc_gather(pt, gidx)

    patches_out = out.reshape(3, _KER, _KER, B, _TOP).transpose(3, 4, 0, 1, 2)
    mask = mask3.reshape(B, _TOP)
    coord_idx = coord3.transpose(0, 2, 1)
    return patches_out, mask, coord_idx


# speedup vs baseline: 16.7806x; 1.3148x over previous
import functools

import jax
import jax.numpy as jnp
import numpy as np
from jax import lax
from jax.experimental import pallas as pl
from jax.experimental.pallas import tpu as pltpu
from jax.experimental.pallas import tpu_sc as plsc

_PATCH = 16
_KER = 22
_PAD = 6
_NP = 32
_L = _NP * _NP
_TOP = 512
_D = 3 * _KER * _KER
_MIN_OBJ = -1000000.0


_CHUNK = _NP * 484
_FT = _CHUNK // 128

def _unfold_index_tables():
    n = np.arange(_CHUNK)
    pw = n // 484
    r = n % 484
    kh = r // _KER
    j = r % _KER
    col = pw * _PATCH + j - _PAD
    rows = np.where(col < 0, 1000, kh + 2)
    cols = np.clip(col, 0, 511)
    return rows.astype(np.int32), cols.astype(np.int32)

_IDXR_NP, _IDXC_NP = _unfold_index_tables()


def _sc_unfold_body(x_hbm, idxr_hbm, idxc_hbm, out_hbm, stage_v, outb_v,
                    idxr_v, idxc_v):
    b = lax.axis_index("s")
    half = lax.axis_index("c")
    tr = b // 8
    br = b % 8
    pltpu.sync_copy(idxr_hbm, idxr_v)
    pltpu.sync_copy(idxc_hbm, idxc_v)
    zero16 = jnp.zeros((16,), jnp.float32)
    for rr in range(24, 32):
        for q in range(512 // 16):
            stage_v[rr, pl.ds(q * 16, 16)] = zero16

    def body(k, carry):
        c = k // 16
        ph = half * 16 + (k % 16)
        start = jnp.where(ph == 0, 0, ph * _PATCH - 8)
        adj = jnp.where(ph == 0, 8, 0).astype(jnp.int32)
        pltpu.sync_copy(x_hbm.at[b, c, pl.ds(start, 24)],
                        stage_v.at[pl.ds(0, 24)])
        for n in range(_CHUNK // 16):
            rv = idxr_v[pl.ds(n * 16, 16)] - adj
            rv = jnp.where((rv < 0) | (rv > 23), 24, rv)
            cv = idxc_v[pl.ds(n * 16, 16)]
            outb_v[n // 8, pl.ds((n % 8) * 16, 16)] = plsc.load_gather(
                stage_v, [rv, cv])
        pltpu.sync_copy(outb_v, out_hbm.at[c, tr, pl.ds(ph * _FT, _FT), br])
        return carry

    lax.fori_loop(0, 48, body, 0)


_sc_unfold = functools.partial(
    pl.kernel,
    mesh=plsc.VectorSubcoreMesh(core_axis_name="c", subcore_axis_name="s"),
    out_type=jax.ShapeDtypeStruct((3, 2, 3872, 8, 128), jnp.float32),
    scratch_types=[
        pltpu.VMEM((32, 512), jnp.float32),
        pltpu.VMEM((_FT, 128), jnp.float32),
        pltpu.VMEM((_CHUNK,), jnp.int32),
        pltpu.VMEM((_CHUNK,), jnp.int32),
    ],
    compiler_params=pltpu.CompilerParams(needs_layout_passes=False),
)(_sc_unfold_body)


def _sort_body(sums_ref, idx_ref, gidx_ref, mask_ref, coord_ref):
    b = pl.program_id(0)
    s = sums_ref[0, 0, :]
    si = s[:, None]
    sj = s[None, :]
    ii = lax.broadcasted_iota(jnp.int32, (_L, _L), 0)
    jj = lax.broadcasted_iota(jnp.int32, (_L, _L), 1)
    before = (sj > si) | ((sj == si) & (jj < ii))
    rank = jnp.sum(before.astype(jnp.float32), axis=1).astype(jnp.int32)
    onehot = rank[:, None] == jj
    sidx_full = jnp.sum(jnp.where(onehot, ii, 0), axis=0)
    ssum_full = jnp.sum(jnp.where(onehot, s[:, None], 0.0), axis=0)
    sidx = sidx_full[:_TOP]
    ssum = ssum_full[:_TOP]
    idx_ref[0, 0, :] = sidx
    gidx_ref[0, 0, :] = sidx + b * _L
    mask_ref[0, 0, :] = (ssum > _MIN_OBJ).astype(jnp.float32)
    coord_ref[0, 0, :] = sidx // _NP
    coord_ref[0, 1, :] = sidx % _NP


_sort_call = pl.pallas_call(
    _sort_body,
    grid=(16,),
    in_specs=[pl.BlockSpec((1, 1, _L), lambda b: (b, 0, 0))],
    out_specs=[
        pl.BlockSpec((1, 1, _TOP), lambda b: (b, 0, 0)),
        pl.BlockSpec((1, 1, _TOP), lambda b: (b, 0, 0)),
        pl.BlockSpec((1, 1, _TOP), lambda b: (b, 0, 0)),
        pl.BlockSpec((1, 2, _TOP), lambda b: (b, 0, 0)),
    ],
    out_shape=[
        jax.ShapeDtypeStruct((16, 1, _TOP), jnp.int32),
        jax.ShapeDtypeStruct((16, 1, _TOP), jnp.int32),
        jax.ShapeDtypeStruct((16, 1, _TOP), jnp.float32),
        jax.ShapeDtypeStruct((16, 2, _TOP), jnp.int32),
    ],
)


_NW = 32
_FPW = (_D + _NW - 1) // _NW
_B16 = 16 * _L
_OUTW = 16 * _TOP


def _sc_gather_body(pt_hbm, gidx_hbm, out_hbm, plane_v, out_v, idx_v):
    wid = lax.axis_index("s") * 2 + lax.axis_index("c")
    pltpu.sync_copy(gidx_hbm, idx_v)
    lo = wid * _FPW

    def body(k, carry):
        f = lo + k

        @pl.when(f < _D)
        def _():
            pltpu.sync_copy(pt_hbm.at[f], plane_v)
            for ch in range(_OUTW // 16):
                ip = idx_v[ch // 8, pl.ds((ch % 8) * 16, 16)]
                hi = lax.shift_right_logical(ip, 7)
                lo_ = lax.bitwise_and(ip, 127)
                vals = plsc.load_gather(plane_v, [hi, lo_])
                out_v[ch // 8, pl.ds((ch % 8) * 16, 16)] = vals
            pltpu.sync_copy(out_v, out_hbm.at[f])

        return carry

    lax.fori_loop(0, _FPW, body, 0)


_sc_gather = functools.partial(
    pl.kernel,
    mesh=plsc.VectorSubcoreMesh(core_axis_name="c", subcore_axis_name="s"),
    out_type=jax.ShapeDtypeStruct((_D, _OUTW // 128, 128), jnp.float32),
    scratch_types=[
        pltpu.VMEM((_B16 // 128, 128), jnp.float32),
        pltpu.VMEM((_OUTW // 128, 128), jnp.float32),
        pltpu.VMEM((_OUTW // 128, 128), jnp.int32),
    ],
    compiler_params=pltpu.CompilerParams(needs_layout_passes=False),
)(_sc_gather_body)


def kernel(x):
    B, C, H, W = x.shape
    out5 = _sc_unfold(x, jnp.asarray(_IDXR_NP), jnp.asarray(_IDXC_NP))
    u = out5.transpose(0, 1, 3, 2, 4).reshape(3, B, _L * 484)
    patches = (u.reshape(3, B, _NP, _NP, _KER, _KER)
                .transpose(1, 2, 3, 0, 4, 5).reshape(B, _L, _D))
    sums = patches.sum(axis=-1)

    pt = (u.reshape(3, B, _L, 484).transpose(0, 3, 1, 2)
           .reshape(_D, _B16 // 128, 128))

    sidx3, gidx3, mask3, coord3 = _sort_call(sums.reshape(B, 1, _L))
    gidx = gidx3.reshape(_OUTW // 128, 128)

    out = _sc_gather(pt, gidx)

    patches_out = out.reshape(3, _KER, _KER, B, _TOP).transpose(3, 4, 0, 1, 2)
    mask = mask3.reshape(B, _TOP)
    coord_idx = coord3.transpose(0, 2, 1)
    return patches_out, mask, coord_idx
